# baseline (speedup 1.0000x reference)
import functools

import jax
import jax.numpy as jnp
from jax import lax
from jax.experimental import pallas as pl
from jax.experimental.pallas import tpu as pltpu
from jax.experimental.pallas import tpu_sc as plsc

_B = 16384
_V = 1000
_NC = 2
_NS = 16
_L = 16
_NW = _NC * _NS
_RPW = _B // _NW
_CH = _L
_NCH = _RPW // _CH

_LN2 = 0.6931471805599453


def _ln(s):
    bits = plsc.bitcast(s, jnp.int32)
    e = ((bits >> 23) & 0xFF) - 127
    f = plsc.bitcast((bits & 0x7FFFFF) | 0x3F800000, jnp.float32)
    t = (f - 1.0) / (f + 1.0)
    t2 = t * t
    p = jnp.float32(1.0 / 9.0) * t2 + jnp.float32(1.0 / 7.0)
    p = p * t2 + jnp.float32(1.0 / 5.0)
    p = p * t2 + jnp.float32(1.0 / 3.0)
    p = p * t2 + 1.0
    return e.astype(jnp.float32) * _LN2 + 2.0 * t * p


def _sc_body(y1_hbm, y2_hbm, t_hbm, out_hbm,
             b1a, b2a, b1b, b2b, tbuf, res, sema, semb):
    wid = lax.axis_index("c") * _NS + lax.axis_index("s")
    row0 = wid * _RPW
    pltpu.sync_copy(t_hbm.at[pl.ds(row0, _RPW)], tbuf)
    lanes = lax.iota(jnp.int32, _L)

    def start(c, b1, b2, sem):
        base = row0 + c * _CH
        pltpu.async_copy(y1_hbm.at[pl.ds(base, _CH)], b1, sem)
        pltpu.async_copy(y2_hbm.at[pl.ds(base, _CH)], b2, sem)

    def wait(b1, b2, sem):
        pltpu.make_async_copy(y1_hbm.at[pl.ds(0, _CH)], b1, sem).wait()
        pltpu.make_async_copy(y2_hbm.at[pl.ds(0, _CH)], b2, sem).wait()

    def compute(c, b1, b2, accs):
        acc_n, acc1n, acc2n, acc1a, acc2a = accs

        ninf = jnp.full((_L,), -jnp.inf, jnp.float32)
        zf = jnp.zeros((_L,), jnp.float32)
        zi = jnp.zeros((_L,), jnp.int32)

        @plsc.parallel_loop(0, _V - 24, unroll=16,
                            carry=(lanes, ninf, zi, zf, ninf, zi, zf))
        def col_loop(j, carry):
            jv, m1, a1, s1, m2, a2, s2 = carry
            x1 = plsc.load_gather(b1, [lanes, jv])
            x2 = plsc.load_gather(b2, [lanes, jv])
            g1 = x1 > m1
            m1 = jnp.where(g1, x1, m1)
            a1 = jnp.where(g1, jv, a1)
            s1 = s1 + jnp.exp(x1)
            g2 = x2 > m2
            m2 = jnp.where(g2, x2, m2)
            a2 = jnp.where(g2, jv, a2)
            s2 = s2 + jnp.exp(x2)
            return (jv + 1, m1, a1, s1, m2, a2, s2)

        @plsc.parallel_loop(_V - 24, _V, unroll=8, carry=col_loop)
        def tail_loop(j, carry):
            jv, m1, a1, s1, m2, a2, s2 = carry
            x1 = plsc.load_gather(b1, [lanes, jv])
            x2 = plsc.load_gather(b2, [lanes, jv])
            g1 = (x1 > m1) | ((x1 == m1) & (jv < a1))
            m1 = jnp.maximum(m1, x1)
            a1 = jnp.where(g1, jv, a1)
            s1 = s1 + jnp.exp(x1)
            g2 = (x2 > m2) | ((x2 == m2) & (jv < a2))
            m2 = jnp.maximum(m2, x2)
            a2 = jnp.where(g2, jv, a2)
            s2 = s2 + jnp.exp(x2)
            jvn = jv + 1
            jv = jnp.where(jvn == _V, 0, jvn)
            return (jv, m1, a1, s1, m2, a2, s2)

        _, m1, a1, s1, m2, a2, s2 = tail_loop

        tv = tbuf[pl.ds(c * _CH, _CH)]
        p1 = plsc.load_gather(b1, [lanes, tv])
        p2 = plsc.load_gather(b2, [lanes, tv])
        ce1 = _ln(s1) - p1
        ce2 = _ln(s2) - p2
        neq = a1 != a2
        zero = jnp.zeros((_L,), jnp.float32)
        return (acc_n + jnp.where(neq, 1.0, 0.0),
                acc1n + jnp.where(neq, ce1, zero),
                acc2n + jnp.where(neq, ce2, zero),
                acc1a + ce1,
                acc2a + ce2)

    start(0, b1a, b2a, sema)
    start(1, b1b, b2b, semb)

    def pair_body(i, accs):
        c = i * 2
        wait(b1a, b2a, sema)
        accs = compute(c, b1a, b2a, accs)

        @pl.when(c + 2 < _NCH)
        def _():
            start(c + 2, b1a, b2a, sema)

        wait(b1b, b2b, semb)
        accs = compute(c + 1, b1b, b2b, accs)

        @pl.when(c + 3 < _NCH)
        def _():
            start(c + 3, b1b, b2b, semb)

        return accs

    z = jnp.zeros((_L,), jnp.float32)
    acc_n, acc1n, acc2n, acc1a, acc2a = lax.fori_loop(
        0, _NCH // 2, pair_body, (z, z, z, z, z))
    res[0, :] = acc_n
    res[1, :] = acc1n
    res[2, :] = acc2n
    res[3, :] = acc1a
    res[4, :] = acc2a
    pltpu.sync_copy(res, out_hbm.at[wid])


@jax.jit
def _sc_partials(y_1, y_2, targets):
    mesh = plsc.VectorSubcoreMesh(core_axis_name="c", subcore_axis_name="s")
    run = pl.kernel(
        _sc_body,
        out_type=jax.ShapeDtypeStruct((_NW, 5, _L), jnp.float32),
        mesh=mesh,
        compiler_params=pltpu.CompilerParams(needs_layout_passes=False),
        scratch_types=[
            pltpu.VMEM((_CH, _V), jnp.float32),
            pltpu.VMEM((_CH, _V), jnp.float32),
            pltpu.VMEM((_CH, _V), jnp.float32),
            pltpu.VMEM((_CH, _V), jnp.float32),
            pltpu.VMEM((_RPW,), jnp.int32),
            pltpu.VMEM((5, _L), jnp.float32),
            pltpu.SemaphoreType.DMA,
            pltpu.SemaphoreType.DMA,
        ],
    )
    return run(y_1, y_2, targets)


def kernel(y_1, y_2, targets, keep_rate=1):
    parts = _sc_partials(y_1, y_2, targets)
    sums = jnp.sum(parts, axis=(0, 2))
    n = sums[0]
    no_dis = n < 0.5
    cnt = jnp.where(no_dis, jnp.float32(_B), n)
    m1 = jnp.where(no_dis, sums[3], sums[1]) / cnt
    m2 = jnp.where(no_dis, sums[4], sums[2]) / cnt
    return (m1, m2, m1, m2)


# speedup vs baseline: 1.1599x; 1.1599x over previous
import functools

import jax
import jax.numpy as jnp
from jax import lax
from jax.experimental import pallas as pl
from jax.experimental.pallas import tpu as pltpu
from jax.experimental.pallas import tpu_sc as plsc

_B = 16384
_V = 1000
_NC = 2
_NS = 16
_L = 16
_NW = _NC * _NS
_RPW = _B // _NW
_CH = _L
_NCH = _RPW // _CH

_LN2 = 0.6931471805599453


def _ln(s):
    bits = plsc.bitcast(s, jnp.int32)
    e = ((bits >> 23) & 0xFF) - 127
    f = plsc.bitcast((bits & 0x7FFFFF) | 0x3F800000, jnp.float32)
    t = (f - 1.0) / (f + 1.0)
    t2 = t * t
    p = jnp.float32(1.0 / 9.0) * t2 + jnp.float32(1.0 / 7.0)
    p = p * t2 + jnp.float32(1.0 / 5.0)
    p = p * t2 + jnp.float32(1.0 / 3.0)
    p = p * t2 + 1.0
    return e.astype(jnp.float32) * _LN2 + 2.0 * t * p


def _sc_body(y1_hbm, y2_hbm, t_hbm, out_hbm,
             b1a, b2a, b1b, b2b, tbuf, res, sb1, sb2, sema, semb):
    wid = lax.axis_index("c") * _NS + lax.axis_index("s")
    row0 = wid * _RPW
    pltpu.sync_copy(t_hbm.at[pl.ds(row0, _RPW)], tbuf)
    lanes = lax.iota(jnp.int32, _L)

    def start(c, b1, b2, sem):
        base = row0 + c * _CH
        pltpu.async_copy(y1_hbm.at[pl.ds(base, _CH)], b1, sem)
        pltpu.async_copy(y2_hbm.at[pl.ds(base, _CH)], b2, sem)

    def wait(b1, b2, sem):
        pltpu.make_async_copy(y1_hbm.at[pl.ds(0, _CH)], b1, sem).wait()
        pltpu.make_async_copy(y2_hbm.at[pl.ds(0, _CH)], b2, sem).wait()

    def compute(c, b1, b2, accs):
        acc_n, acc1n, acc2n, acc1a, acc2a = accs

        ninf = jnp.full((_L,), -jnp.inf, jnp.float32)
        zf = jnp.zeros((_L,), jnp.float32)
        zi = jnp.zeros((_L,), jnp.int32)
        sb1[...] = zf
        sb2[...] = zf

        @plsc.parallel_loop(0, _V - 16, unroll=8,
                            carry=(lanes, ninf, zi, ninf, zi))
        def col_loop(j, carry):
            jv, m1, a1, m2, a2 = carry
            x1 = plsc.load_gather(b1, [lanes, jv])
            x2 = plsc.load_gather(b2, [lanes, jv])
            g1 = x1 > m1
            m1 = jnp.where(g1, x1, m1)
            a1 = jnp.where(g1, jv, a1)
            plsc.addupdate(sb1.at[...], jnp.exp(x1))
            g2 = x2 > m2
            m2 = jnp.where(g2, x2, m2)
            a2 = jnp.where(g2, jv, a2)
            plsc.addupdate(sb2.at[...], jnp.exp(x2))
            return (jv + 1, m1, a1, m2, a2)

        @plsc.parallel_loop(_V - 16, _V, unroll=8, carry=col_loop)
        def tail_loop(j, carry):
            jv, m1, a1, m2, a2 = carry
            x1 = plsc.load_gather(b1, [lanes, jv])
            x2 = plsc.load_gather(b2, [lanes, jv])
            g1 = (x1 > m1) | ((x1 == m1) & (jv < a1))
            m1 = jnp.maximum(m1, x1)
            a1 = jnp.where(g1, jv, a1)
            plsc.addupdate(sb1.at[...], jnp.exp(x1))
            g2 = (x2 > m2) | ((x2 == m2) & (jv < a2))
            m2 = jnp.maximum(m2, x2)
            a2 = jnp.where(g2, jv, a2)
            plsc.addupdate(sb2.at[...], jnp.exp(x2))
            jvn = jv + 1
            jv = jnp.where(jvn == _V, 0, jvn)
            return (jv, m1, a1, m2, a2)

        _, m1, a1, m2, a2 = tail_loop
        s1 = sb1[...]
        s2 = sb2[...]

        tv = tbuf[pl.ds(c * _CH, _CH)]
        p1 = plsc.load_gather(b1, [lanes, tv])
        p2 = plsc.load_gather(b2, [lanes, tv])
        ce1 = _ln(s1) - p1
        ce2 = _ln(s2) - p2
        neq = a1 != a2
        zero = jnp.zeros((_L,), jnp.float32)
        return (acc_n + jnp.where(neq, 1.0, 0.0),
                acc1n + jnp.where(neq, ce1, zero),
                acc2n + jnp.where(neq, ce2, zero),
                acc1a + ce1,
                acc2a + ce2)

    start(0, b1a, b2a, sema)
    start(1, b1b, b2b, semb)

    def pair_body(i, accs):
        c = i * 2
        wait(b1a, b2a, sema)
        accs = compute(c, b1a, b2a, accs)

        @pl.when(c + 2 < _NCH)
        def _():
            start(c + 2, b1a, b2a, sema)

        wait(b1b, b2b, semb)
        accs = compute(c + 1, b1b, b2b, accs)

        @pl.when(c + 3 < _NCH)
        def _():
            start(c + 3, b1b, b2b, semb)

        return accs

    z = jnp.zeros((_L,), jnp.float32)
    acc_n, acc1n, acc2n, acc1a, acc2a = lax.fori_loop(
        0, _NCH // 2, pair_body, (z, z, z, z, z))
    res[0, :] = acc_n
    res[1, :] = acc1n
    res[2, :] = acc2n
    res[3, :] = acc1a
    res[4, :] = acc2a
    pltpu.sync_copy(res, out_hbm.at[wid])


@jax.jit
def _sc_partials(y_1, y_2, targets):
    mesh = plsc.VectorSubcoreMesh(core_axis_name="c", subcore_axis_name="s")
    run = pl.kernel(
        _sc_body,
        out_type=jax.ShapeDtypeStruct((_NW, 5, _L), jnp.float32),
        mesh=mesh,
        compiler_params=pltpu.CompilerParams(needs_layout_passes=False),
        scratch_types=[
            pltpu.VMEM((_CH, _V), jnp.float32),
            pltpu.VMEM((_CH, _V), jnp.float32),
            pltpu.VMEM((_CH, _V), jnp.float32),
            pltpu.VMEM((_CH, _V), jnp.float32),
            pltpu.VMEM((_RPW,), jnp.int32),
            pltpu.VMEM((5, _L), jnp.float32),
            pltpu.VMEM((_L,), jnp.float32),
            pltpu.VMEM((_L,), jnp.float32),
            pltpu.SemaphoreType.DMA,
            pltpu.SemaphoreType.DMA,
        ],
    )
    return run(y_1, y_2, targets)


def kernel(y_1, y_2, targets, keep_rate=1):
    parts = _sc_partials(y_1, y_2, targets)
    sums = jnp.sum(parts, axis=(0, 2))
    n = sums[0]
    no_dis = n < 0.5
    cnt = jnp.where(no_dis, jnp.float32(_B), n)
    m1 = jnp.where(no_dis, sums[3], sums[1]) / cnt
    m2 = jnp.where(no_dis, sums[4], sums[2]) / cnt
    return (m1, m2, m1, m2)


# speedup vs baseline: 1.1606x; 1.0006x over previous
import jax
import jax.numpy as jnp
from jax import lax
from jax.experimental import pallas as pl
from jax.experimental.pallas import tpu as pltpu
from jax.experimental.pallas import tpu_sc as plsc

_B = 16384
_V = 1000
_NC = 2
_NS = 16
_L = 16
_NW = _NC * _NS
_RPW = _B // _NW
_CH = _L
_NCH = _RPW // _CH

_LN2 = 0.6931471805599453


def _ln(s):
    bits = plsc.bitcast(s, jnp.int32)
    e = ((bits >> 23) & 0xFF) - 127
    f = plsc.bitcast((bits & 0x7FFFFF) | 0x3F800000, jnp.float32)
    t = (f - 1.0) / (f + 1.0)
    t2 = t * t
    p = jnp.float32(1.0 / 9.0) * t2 + jnp.float32(1.0 / 7.0)
    p = p * t2 + jnp.float32(1.0 / 5.0)
    p = p * t2 + jnp.float32(1.0 / 3.0)
    p = p * t2 + 1.0
    return e.astype(jnp.float32) * _LN2 + 2.0 * t * p


def _sc_body(y1_hbm, y2_hbm, t_hbm, out_hbm,
             b1a, b2a, b1b, b2b, tbuf, res, sb1, sb2, sema, semb):
    wid = lax.axis_index("c") * _NS + lax.axis_index("s")
    row0 = wid * _RPW
    pltpu.sync_copy(t_hbm.at[pl.ds(row0, _RPW)], tbuf)
    lanes = lax.iota(jnp.int32, _L)

    def start(c, b1, b2, sem):
        base = row0 + c * _CH
        pltpu.async_copy(y1_hbm.at[pl.ds(base, _CH)], b1, sem)
        pltpu.async_copy(y2_hbm.at[pl.ds(base, _CH)], b2, sem)

    def wait(b1, b2, sem):
        pltpu.make_async_copy(y1_hbm.at[pl.ds(0, _CH)], b1, sem).wait()
        pltpu.make_async_copy(y2_hbm.at[pl.ds(0, _CH)], b2, sem).wait()

    def compute(c, b1, b2, accs):
        acc_n, acc1n, acc2n, acc1a, acc2a = accs

        ninf = jnp.full((_L,), -jnp.inf, jnp.float32)
        zf = jnp.zeros((_L,), jnp.float32)
        zi = jnp.zeros((_L,), jnp.int32)
        sb1[...] = zf
        sb2[...] = zf

        @plsc.parallel_loop(0, _V - 16, unroll=8,
                            carry=(lanes, ninf, zi, ninf, zi))
        def col_loop(j, carry):
            jv, m1, a1, m2, a2 = carry
            x1 = plsc.load_gather(b1, [lanes, jv])
            x2 = plsc.load_gather(b2, [lanes, jv])
            g1 = x1 > m1
            m1 = jnp.where(g1, x1, m1)
            a1 = jnp.where(g1, jv, a1)
            plsc.addupdate(sb1.at[...], jnp.exp(x1))
            g2 = x2 > m2
            m2 = jnp.where(g2, x2, m2)
            a2 = jnp.where(g2, jv, a2)
            plsc.addupdate(sb2.at[...], jnp.exp(x2))
            return (jv + 1, m1, a1, m2, a2)

        @plsc.parallel_loop(_V - 16, _V, unroll=8, carry=col_loop)
        def tail_loop(j, carry):
            jv, m1, a1, m2, a2 = carry
            x1 = plsc.load_gather(b1, [lanes, jv])
            x2 = plsc.load_gather(b2, [lanes, jv])
            g1 = (x1 > m1) | ((x1 == m1) & (jv < a1))
            m1 = jnp.maximum(m1, x1)
            a1 = jnp.where(g1, jv, a1)
            plsc.addupdate(sb1.at[...], jnp.exp(x1))
            g2 = (x2 > m2) | ((x2 == m2) & (jv < a2))
            m2 = jnp.maximum(m2, x2)
            a2 = jnp.where(g2, jv, a2)
            plsc.addupdate(sb2.at[...], jnp.exp(x2))
            jvn = jv + 1
            jv = jnp.where(jvn == _V, 0, jvn)
            return (jv, m1, a1, m2, a2)

        _, m1, a1, m2, a2 = tail_loop
        s1 = sb1[...]
        s2 = sb2[...]

        tv = tbuf[pl.ds(c * _CH, _CH)]
        p1 = plsc.load_gather(b1, [lanes, tv])
        p2 = plsc.load_gather(b2, [lanes, tv])
        ce1 = _ln(s1) - p1
        ce2 = _ln(s2) - p2
        neq = a1 != a2
        zero = jnp.zeros((_L,), jnp.float32)
        return (acc_n + jnp.where(neq, 1.0, 0.0),
                acc1n + jnp.where(neq, ce1, zero),
                acc2n + jnp.where(neq, ce2, zero),
                acc1a + ce1,
                acc2a + ce2)

    start(0, b1a, b2a, sema)
    start(1, b1b, b2b, semb)

    def pair_body(i, accs):
        c = i * 2
        wait(b1a, b2a, sema)
        accs = compute(c, b1a, b2a, accs)

        @pl.when(c + 2 < _NCH)
        def _():
            start(c + 2, b1a, b2a, sema)

        wait(b1b, b2b, semb)
        accs = compute(c + 1, b1b, b2b, accs)

        @pl.when(c + 3 < _NCH)
        def _():
            start(c + 3, b1b, b2b, semb)

        return accs

    z = jnp.zeros((_L,), jnp.float32)
    acc_n, acc1n, acc2n, acc1a, acc2a = lax.fori_loop(
        0, _NCH // 2, pair_body, (z, z, z, z, z))
    res[0, :] = acc_n
    res[1, :] = acc1n
    res[2, :] = acc2n
    res[3, :] = acc1a
    res[4, :] = acc2a
    pltpu.sync_copy(res, out_hbm.at[wid])


@jax.jit
def _sc_partials(y_1, y_2, targets):
    mesh = plsc.VectorSubcoreMesh(core_axis_name="c", subcore_axis_name="s")
    run = pl.kernel(
        _sc_body,
        out_type=jax.ShapeDtypeStruct((_NW, 5, _L), jnp.float32),
        mesh=mesh,
        compiler_params=pltpu.CompilerParams(needs_layout_passes=False),
        scratch_types=[
            pltpu.VMEM((_CH, _V), jnp.float32),
            pltpu.VMEM((_CH, _V), jnp.float32),
            pltpu.VMEM((_CH, _V), jnp.float32),
            pltpu.VMEM((_CH, _V), jnp.float32),
            pltpu.VMEM((_RPW,), jnp.int32),
            pltpu.VMEM((5, _L), jnp.float32),
            pltpu.VMEM((_L,), jnp.float32),
            pltpu.VMEM((_L,), jnp.float32),
            pltpu.SemaphoreType.DMA,
            pltpu.SemaphoreType.DMA,
        ],
    )
    return run(y_1, y_2, targets)


def kernel(y_1, y_2, targets, keep_rate=1):
    parts = _sc_partials(y_1, y_2, targets)
    sums = jnp.sum(parts, axis=(0, 2))
    n = sums[0]
    no_dis = n < 0.5
    cnt = jnp.where(no_dis, jnp.float32(_B), n)
    m1 = jnp.where(no_dis, sums[3], sums[1]) / cnt
    m2 = jnp.where(no_dis, sums[4], sums[2]) / cnt
    return (m1, m2, m1, m2)
